# Initial kernel scaffold; baseline (speedup 1.0000x reference)
import jax
import jax.numpy as jnp
from jax.experimental import pallas as pl


def kernel(pred):
    raise NotImplementedError("write your pallas kernel here")


# speedup vs baseline: 1.3984x; 1.3984x over previous
import jax
import jax.numpy as jnp
from jax import lax
from jax.experimental import pallas as pl
from jax.experimental.pallas import tpu as pltpu

_N, _NCLS = 5000, 80
_NP = 5120
_KP = 1024
_PRE_K = 1000
_MAXD = 300
_CONF_T, _IOU_T = 0.25, 0.45
_HI = lax.Precision.HIGHEST
_C = 128


def _row_of(col):
    n = col.shape[0]
    eye = (lax.broadcasted_iota(jnp.int32, (_C, _C), 0)
           == lax.broadcasted_iota(jnp.int32, (_C, _C), 1)).astype(jnp.float32)
    outs = []
    for c in range(n // _C):
        xc = lax.slice(col, (c * _C, 0), ((c + 1) * _C, 1))
        outs.append(jnp.sum(xc * eye, axis=0, keepdims=True))
    return jnp.concatenate(outs, axis=1)


def _col_of(row):
    n = row.shape[1]
    eye = (lax.broadcasted_iota(jnp.int32, (_C, _C), 0)
           == lax.broadcasted_iota(jnp.int32, (_C, _C), 1)).astype(jnp.float32)
    outs = []
    for c in range(n // _C):
        rc = lax.slice(row, (0, c * _C), (1, (c + 1) * _C))
        outs.append(jnp.sum(rc * eye, axis=1, keepdims=True))
    return jnp.concatenate(outs, axis=0)


def _ranks_looped(col_ref, rank_ref, m):
    eye = (lax.broadcasted_iota(jnp.int32, (_C, _C), 0)
           == lax.broadcasted_iota(jnp.int32, (_C, _C), 1)).astype(jnp.float32)
    col = col_ref[0:m, :]
    i_sub = lax.broadcasted_iota(jnp.int32, (m, _C), 0)
    j_lane = lax.broadcasted_iota(jnp.int32, (m, _C), 1)
    rank_ref[0:m, :] = jnp.zeros((m, 1), jnp.float32)

    def body(c, _):
        off = pl.multiple_of(c * _C, _C)
        cc = col_ref[pl.ds(off, _C), :]
        rc = jnp.sum(cc * eye, axis=0, keepdims=True)
        gt = (rc > col).astype(jnp.float32)
        jj = j_lane + c * _C
        tie = ((rc == col) & (jj < i_sub)).astype(jnp.float32)
        rank_ref[0:m, :] = rank_ref[0:m, :] + jnp.sum(gt + tie, axis=1, keepdims=True)
        return 0

    lax.fori_loop(0, m // _C, body, 0)


def _body(pred_ref, out_ref, m_ref, cc_ref, rk_ref):
    pred = pred_ref[0]
    obj = pred[:, 4:5]
    scores = pred[:, 5:] * obj
    conf0 = jnp.max(scores, axis=1, keepdims=True)
    cl_iota = lax.broadcasted_iota(jnp.int32, (_N, _NCLS), 1).astype(jnp.float32)
    cls = jnp.min(jnp.where(scores == conf0, cl_iota, 1e9), axis=1, keepdims=True)
    conf = jnp.where(conf0 > _CONF_T, conf0, 0.0)

    confp = jnp.concatenate(
        [conf, jnp.full((_NP - _N, 1), -1.0, jnp.float32)], axis=0)
    cc_ref[:, :] = confp
    _ranks_looped(cc_ref, rk_ref, _NP)
    rank = rk_ref[:, :]
    rank_row = _row_of(rank)

    xy = pred[:, 0:2]
    wh = pred[:, 2:4]
    boxes = jnp.concatenate([xy - wh / 2.0, xy + wh / 2.0], axis=1)
    vals = jnp.concatenate([boxes, conf, cls], axis=1)
    valsp = jnp.concatenate(
        [vals, jnp.zeros((_NP - _N, 6), jnp.float32)], axis=0)

    tb = []
    for kb in range(_KP // _C):
        ki = float(kb * _C) + lax.broadcasted_iota(jnp.int32, (_C, _NP), 0).astype(jnp.float32)
        p = (rank_row == ki).astype(jnp.float32)
        tb.append(jnp.dot(p, valsp, precision=_HI))
    topb = jnp.concatenate(tb, axis=0)

    clsc = topb[:, 5:6]
    off = clsc * 4096.0
    x1c = topb[:, 0:1] + off
    y1c = topb[:, 1:2] + off
    x2c = topb[:, 2:3] + off
    y2c = topb[:, 3:4] + off
    areac = (x2c - x1c) * (y2c - y1c)
    x1r, y1r = _row_of(x1c), _row_of(y1c)
    x2r, y2r = _row_of(x2c), _row_of(y2c)
    arear = _row_of(areac)

    for mb in range(_KP // _C):
        s = mb * _C
        x1b = lax.slice(x1c, (s, 0), (s + _C, 1))
        y1b = lax.slice(y1c, (s, 0), (s + _C, 1))
        x2b = lax.slice(x2c, (s, 0), (s + _C, 1))
        y2b = lax.slice(y2c, (s, 0), (s + _C, 1))
        arb = lax.slice(areac, (s, 0), (s + _C, 1))
        ltx = jnp.maximum(x1b, x1r)
        lty = jnp.maximum(y1b, y1r)
        rbx = jnp.minimum(x2b, x2r)
        rby = jnp.minimum(y2b, y2r)
        w = jnp.clip(rbx - ltx, 0.0)
        h = jnp.clip(rby - lty, 0.0)
        inter = w * h
        iou = inter / (arb + arear - inter + 1e-7)
        jl = lax.broadcasted_iota(jnp.int32, (_C, _KP), 1).astype(jnp.float32)
        ii = float(s) + lax.broadcasted_iota(jnp.int32, (_C, _KP), 0).astype(jnp.float32)
        mbv = ((iou > _IOU_T) & (jl > ii)).astype(jnp.float32)
        m_ref[s:s + _C, :] = mbv

    topvc = topb[:, 4:5]
    topv_row = _row_of(topvc)
    klane = lax.broadcasted_iota(jnp.int32, (1, _KP), 1)
    keep0 = ((topv_row > 0.0)
             & (klane < _PRE_K)).astype(jnp.float32)

    def nms_step(i, keep):
        mrow = m_ref[pl.ds(i, 1), :]
        sel = (klane == i).astype(jnp.float32)
        ki = jnp.sum(keep * sel)
        return keep * (1.0 - mrow * ki)

    keep = lax.fori_loop(0, _PRE_K, nms_step, keep0)

    fsr = keep * topv_row
    fsc = _col_of(fsr)
    cc_ref[0:_KP, :] = fsc
    _ranks_looped(cc_ref, rk_ref, _KP)
    frank = rk_ref[0:_KP, :]
    frank_row = _row_of(frank)
    vals_k = jnp.concatenate([topb[:, 0:4], fsc, clsc], axis=1)
    outs = []
    for fb in range(3):
        ki = float(fb * _C) + lax.broadcasted_iota(jnp.int32, (_C, _KP), 0).astype(jnp.float32)
        p = (frank_row == ki).astype(jnp.float32)
        oc = jnp.dot(p, vals_k, precision=_HI)
        fvc = jnp.dot(p, fsc, precision=_HI)
        outs.append(oc * (fvc > 0.0).astype(jnp.float32))
    res = jnp.concatenate(outs, axis=0)
    out_ref[0] = lax.slice(res, (0, 0), (_MAXD, 6))


def kernel(pred):
    return pl.pallas_call(
        _body,
        grid=(8,),
        in_specs=[pl.BlockSpec((1, _N, 85), lambda i: (i, 0, 0))],
        out_specs=pl.BlockSpec((1, _MAXD, 6), lambda i: (i, 0, 0)),
        out_shape=jax.ShapeDtypeStruct((8, _MAXD, 6), jnp.float32),
        scratch_shapes=[pltpu.VMEM((_KP, _KP), jnp.float32),
                        pltpu.VMEM((_NP, 1), jnp.float32),
                        pltpu.VMEM((_NP, 1), jnp.float32)],
    )(pred)


# speedup vs baseline: 3.0669x; 2.1931x over previous
import functools
import jax
import jax.numpy as jnp
from jax import lax
from jax.experimental import pallas as pl
from jax.experimental.pallas import tpu as pltpu
from jax.experimental.pallas import tpu_sc as plsc

_N, _NCLS = 5000, 80
_NP = 5120
_KP = 1024
_PRE_K = 1000
_MAXD = 300
_CONF_T, _IOU_T = 0.25, 0.45
_HI = lax.Precision.HIGHEST
_C = 128
_W = _KP // 16


def _row_of(col):
    n = col.shape[0]
    eye = (lax.broadcasted_iota(jnp.int32, (_C, _C), 0)
           == lax.broadcasted_iota(jnp.int32, (_C, _C), 1)).astype(jnp.float32)
    outs = []
    for c in range(n // _C):
        xc = lax.slice(col, (c * _C, 0), ((c + 1) * _C, 1))
        outs.append(jnp.sum(xc * eye, axis=0, keepdims=True))
    return jnp.concatenate(outs, axis=1)


def _col_of(row):
    n = row.shape[1]
    eye = (lax.broadcasted_iota(jnp.int32, (_C, _C), 0)
           == lax.broadcasted_iota(jnp.int32, (_C, _C), 1)).astype(jnp.float32)
    outs = []
    for c in range(n // _C):
        rc = lax.slice(row, (0, c * _C), (1, (c + 1) * _C))
        outs.append(jnp.sum(rc * eye, axis=1, keepdims=True))
    return jnp.concatenate(outs, axis=0)


def _ranks_looped(col_ref, rank_ref, m):
    eye = (lax.broadcasted_iota(jnp.int32, (_C, _C), 0)
           == lax.broadcasted_iota(jnp.int32, (_C, _C), 1)).astype(jnp.float32)
    col = col_ref[0:m, :]
    i_sub = lax.broadcasted_iota(jnp.int32, (m, _C), 0)
    j_lane = lax.broadcasted_iota(jnp.int32, (m, _C), 1)
    rank_ref[0:m, :] = jnp.zeros((m, 1), jnp.float32)

    def body(c, _):
        off = pl.multiple_of(c * _C, _C)
        cc = col_ref[pl.ds(off, _C), :]
        rc = jnp.sum(cc * eye, axis=0, keepdims=True)
        gt = (rc > col).astype(jnp.float32)
        jj = j_lane + c * _C
        tie = ((rc == col) & (jj < i_sub)).astype(jnp.float32)
        rank_ref[0:m, :] = rank_ref[0:m, :] + jnp.sum(gt + tie, axis=1, keepdims=True)
        return 0

    lax.fori_loop(0, m // _C, body, 0)


def _pack_matrix():
    js = lax.broadcasted_iota(jnp.int32, (_KP, 1), 0)
    wl = lax.broadcasted_iota(jnp.int32, (1, _W), 1)
    pw = lax.bitcast_convert_type(((js & 15) + 127) << 23, jnp.float32)
    return jnp.where((js >> 4) == wl, pw, 0.0)


def _tc1_body(pred_ref, topb_ref, mp_ref, ki_ref, cc_ref, rk_ref):
    pred = pred_ref[0]
    obj = pred[:, 4:5]
    scores = pred[:, 5:] * obj
    conf0 = jnp.max(scores, axis=1, keepdims=True)
    cl_iota = lax.broadcasted_iota(jnp.int32, (_N, _NCLS), 1).astype(jnp.float32)
    cls = jnp.min(jnp.where(scores == conf0, cl_iota, 1e9), axis=1, keepdims=True)
    conf = jnp.where(conf0 > _CONF_T, conf0, 0.0)

    confp = jnp.concatenate(
        [conf, jnp.full((_NP - _N, 1), -1.0, jnp.float32)], axis=0)
    cc_ref[:, :] = confp
    _ranks_looped(cc_ref, rk_ref, _NP)
    rank_row = _row_of(rk_ref[:, :])

    xy = pred[:, 0:2]
    wh = pred[:, 2:4]
    boxes = jnp.concatenate([xy - wh / 2.0, xy + wh / 2.0], axis=1)
    vals = jnp.concatenate([boxes, conf, cls], axis=1)
    valsp = jnp.concatenate(
        [vals, jnp.zeros((_NP - _N, 6), jnp.float32)], axis=0)

    tb = []
    for kb in range(_KP // _C):
        ki = float(kb * _C) + lax.broadcasted_iota(jnp.int32, (_C, _NP), 0).astype(jnp.float32)
        p = (rank_row == ki).astype(jnp.float32)
        tb.append(jnp.dot(p, valsp, precision=_HI))
    topb = jnp.concatenate(tb, axis=0)
    topb_ref[0] = topb

    clsc = topb[:, 5:6]
    off = clsc * 4096.0
    x1c = topb[:, 0:1] + off
    y1c = topb[:, 1:2] + off
    x2c = topb[:, 2:3] + off
    y2c = topb[:, 3:4] + off
    areac = (x2c - x1c) * (y2c - y1c)
    x1r, y1r = _row_of(x1c), _row_of(y1c)
    x2r, y2r = _row_of(x2c), _row_of(y2c)
    arear = _row_of(areac)
    pack = _pack_matrix()

    for mb in range(_KP // _C):
        s = mb * _C
        x1b = lax.slice(x1c, (s, 0), (s + _C, 1))
        y1b = lax.slice(y1c, (s, 0), (s + _C, 1))
        x2b = lax.slice(x2c, (s, 0), (s + _C, 1))
        y2b = lax.slice(y2c, (s, 0), (s + _C, 1))
        arb = lax.slice(areac, (s, 0), (s + _C, 1))
        ltx = jnp.maximum(x1b, x1r)
        lty = jnp.maximum(y1b, y1r)
        rbx = jnp.minimum(x2b, x2r)
        rby = jnp.minimum(y2b, y2r)
        w = jnp.clip(rbx - ltx, 0.0)
        h = jnp.clip(rby - lty, 0.0)
        inter = w * h
        iou = inter / (arb + arear - inter + 1e-7)
        jl = lax.broadcasted_iota(jnp.int32, (_C, _KP), 1).astype(jnp.float32)
        ii = float(s) + lax.broadcasted_iota(jnp.int32, (_C, _KP), 0).astype(jnp.float32)
        mbv = ((iou > _IOU_T) & (jl > ii)).astype(jnp.float32)
        mp_ref[0, s:s + _C, :] = jnp.dot(mbv, pack, precision=_HI)

    topvc = topb[:, 4:5]
    topv_row = _row_of(topvc)
    klane = lax.broadcasted_iota(jnp.int32, (1, _KP), 1)
    keep0 = ((topv_row > 0.0) & (klane < _PRE_K)).astype(jnp.float32)
    ki_ref[0] = jnp.dot(keep0, pack, precision=_HI)


def _tc1(pred):
    return pl.pallas_call(
        _tc1_body,
        grid=(8,),
        in_specs=[pl.BlockSpec((1, _N, 85), lambda i: (i, 0, 0))],
        out_specs=[
            pl.BlockSpec((1, _KP, 6), lambda i: (i, 0, 0)),
            pl.BlockSpec((1, _KP, _W), lambda i: (i, 0, 0)),
            pl.BlockSpec((1, 1, _W), lambda i: (i, 0, 0)),
        ],
        out_shape=[
            jax.ShapeDtypeStruct((8, _KP, 6), jnp.float32),
            jax.ShapeDtypeStruct((8, _KP, _W), jnp.float32),
            jax.ShapeDtypeStruct((8, 1, _W), jnp.float32),
        ],
        scratch_shapes=[pltpu.VMEM((_NP, 1), jnp.float32),
                        pltpu.VMEM((_NP, 1), jnp.float32)],
    )(pred)


def _sc_nms(mp, ki):
    mesh = plsc.VectorSubcoreMesh(core_axis_name="c", subcore_axis_name="s")

    @functools.partial(
        pl.kernel,
        mesh=mesh,
        out_type=jax.ShapeDtypeStruct((8, _KP), jnp.float32),
        scratch_types=[
            pltpu.VMEM((_KP * _W,), jnp.float32),
            pltpu.VMEM((_W + 16,), jnp.int32),
            pltpu.VMEM((_W,), jnp.float32),
            pltpu.VMEM((_KP,), jnp.float32),
        ],
    )
    def nms(mp_hbm, ki_hbm, keep_hbm, m_v, kw_v, kf_v, ko_v):
        cid = lax.axis_index("c")
        sid = lax.axis_index("s")
        wid = sid * 2 + cid
        lane16 = lax.broadcasted_iota(jnp.int32, (16,), 0)

        @pl.when(wid < 8)
        def _():
            pltpu.sync_copy(mp_hbm.at[wid], m_v)
            pltpu.sync_copy(ki_hbm.at[wid], kf_v)
            for w4 in range(_W // 16):
                kw_v[pl.ds(w4 * 16, 16)] = kf_v[pl.ds(w4 * 16, 16)].astype(jnp.int32)

            def step(i, carry):
                kv = kw_v[pl.ds(i >> 4, 16)]
                bit = (kv[0] >> (i & 15)) & 1

                @pl.when(bit != 0)
                def _s():
                    base = i * _W
                    for w4 in range(_W // 16):
                        mrow = m_v[pl.ds(base + w4 * 16, 16)].astype(jnp.int32)
                        kw_v[pl.ds(w4 * 16, 16)] = (
                            kw_v[pl.ds(w4 * 16, 16)] & (~mrow))

                return carry

            lax.fori_loop(0, _PRE_K, step, 0)

            for w in range(_W):
                word = kw_v[pl.ds(w, 16)][0]
                wv = jnp.full((16,), word, jnp.int32)
                ko_v[pl.ds(w * 16, 16)] = ((wv >> lane16) & 1).astype(jnp.float32)
            pltpu.sync_copy(ko_v, keep_hbm.at[wid])

    return nms(mp, ki)


def _tc2_body(topb_ref, keep_ref, out_ref, cc_ref, rk_ref):
    topb = topb_ref[0]
    keep = keep_ref[0]
    clsc = topb[:, 5:6]
    topvc = topb[:, 4:5]
    topv_row = _row_of(topvc)
    fsr = keep * topv_row
    fsc = _col_of(fsr)
    cc_ref[0:_KP, :] = fsc
    _ranks_looped(cc_ref, rk_ref, _KP)
    frank_row = _row_of(rk_ref[0:_KP, :])
    vals_k = jnp.concatenate([topb[:, 0:4], fsc, clsc], axis=1)
    outs = []
    for fb in range(3):
        ki = float(fb * _C) + lax.broadcasted_iota(jnp.int32, (_C, _KP), 0).astype(jnp.float32)
        p = (frank_row == ki).astype(jnp.float32)
        oc = jnp.dot(p, vals_k, precision=_HI)
        fvc = jnp.dot(p, fsc, precision=_HI)
        outs.append(oc * (fvc > 0.0).astype(jnp.float32))
    res = jnp.concatenate(outs, axis=0)
    out_ref[0] = lax.slice(res, (0, 0), (_MAXD, 6))


def _tc2(topb, keep):
    return pl.pallas_call(
        _tc2_body,
        grid=(8,),
        in_specs=[pl.BlockSpec((1, _KP, 6), lambda i: (i, 0, 0)),
                  pl.BlockSpec((1, 1, _KP), lambda i: (i, 0, 0))],
        out_specs=pl.BlockSpec((1, _MAXD, 6), lambda i: (i, 0, 0)),
        out_shape=jax.ShapeDtypeStruct((8, _MAXD, 6), jnp.float32),
        scratch_shapes=[pltpu.VMEM((_KP, 1), jnp.float32),
                        pltpu.VMEM((_KP, 1), jnp.float32)],
    )(topb, keep)


def kernel(pred):
    topb, mp, ki = _tc1(pred)
    keep = _sc_nms(jnp.reshape(mp, (8, _KP * _W)), jnp.reshape(ki, (8, _W)))
    return _tc2(topb, jnp.reshape(keep, (8, 1, _KP)))


# speedup vs baseline: 4.1238x; 1.3446x over previous
import functools
import jax
import jax.numpy as jnp
from jax import lax
from jax.experimental import pallas as pl
from jax.experimental.pallas import tpu as pltpu
from jax.experimental.pallas import tpu_sc as plsc

_N, _NCLS = 5000, 80
_NP = 5120
_KP = 1024
_PRE_K = 1000
_MAXD = 300
_CONF_T, _IOU_T = 0.25, 0.45
_HI = lax.Precision.HIGHEST
_C = 128
_W = _KP // 16


def _row_of(col):
    n = col.shape[0]
    eye = (lax.broadcasted_iota(jnp.int32, (_C, _C), 0)
           == lax.broadcasted_iota(jnp.int32, (_C, _C), 1)).astype(jnp.float32)
    outs = []
    for c in range(n // _C):
        xc = lax.slice(col, (c * _C, 0), ((c + 1) * _C, 1))
        outs.append(jnp.sum(xc * eye, axis=0, keepdims=True))
    return jnp.concatenate(outs, axis=1)


def _col_of(row):
    n = row.shape[1]
    eye = (lax.broadcasted_iota(jnp.int32, (_C, _C), 0)
           == lax.broadcasted_iota(jnp.int32, (_C, _C), 1)).astype(jnp.float32)
    outs = []
    for c in range(n // _C):
        rc = lax.slice(row, (0, c * _C), (1, (c + 1) * _C))
        outs.append(jnp.sum(rc * eye, axis=1, keepdims=True))
    return jnp.concatenate(outs, axis=0)


def _ranks_looped(col_ref, rank_ref, acc_ref, m):
    eye = (lax.broadcasted_iota(jnp.int32, (_C, _C), 0)
           == lax.broadcasted_iota(jnp.int32, (_C, _C), 1)).astype(jnp.int32)
    key = lax.bitcast_convert_type(col_ref[0:m, :], jnp.int32)
    i_sub = lax.broadcasted_iota(jnp.int32, (m, _C), 0)
    j_lane = lax.broadcasted_iota(jnp.int32, (m, _C), 1)
    acc_ref[0:m, :] = jnp.zeros((m, _C), jnp.int32)

    def body(c, _):
        off = pl.multiple_of(c * _C, _C)
        kc = lax.bitcast_convert_type(col_ref[pl.ds(off, _C), :], jnp.int32)
        rc = jnp.sum(kc * eye, axis=0, keepdims=True)
        gt = rc > key
        ge = rc >= key
        jlt = (j_lane + c * _C) < i_sub
        contrib = gt | (ge & jlt)
        acc_ref[0:m, :] = acc_ref[0:m, :] + contrib.astype(jnp.int32)
        return 0

    lax.fori_loop(0, m // _C, body, 0)
    rank_ref[0:m, :] = jnp.sum(
        acc_ref[0:m, :], axis=1, keepdims=True).astype(jnp.float32)


def _pack_matrix():
    js = lax.broadcasted_iota(jnp.int32, (_KP, 1), 0)
    wl = lax.broadcasted_iota(jnp.int32, (1, _W), 1)
    pw = lax.bitcast_convert_type(((js & 15) + 127) << 23, jnp.float32)
    return jnp.where((js >> 4) == wl, pw, 0.0)


def _tc1_body(pred_ref, topb_ref, mp_ref, ki_ref, cc_ref, rk_ref, ac_ref):
    pred = pred_ref[0]
    obj = pred[:, 4:5]
    scores = pred[:, 5:] * obj
    conf0 = jnp.max(scores, axis=1, keepdims=True)
    cl_iota = lax.broadcasted_iota(jnp.int32, (_N, _NCLS), 1).astype(jnp.float32)
    cls = jnp.min(jnp.where(scores == conf0, cl_iota, 1e9), axis=1, keepdims=True)
    conf = jnp.where(conf0 > _CONF_T, conf0, 0.0)

    confp = jnp.concatenate(
        [conf, jnp.full((_NP - _N, 1), -1.0, jnp.float32)], axis=0)
    cc_ref[:, :] = confp
    _ranks_looped(cc_ref, rk_ref, ac_ref, _NP)
    rank_row = _row_of(rk_ref[:, :])

    xy = pred[:, 0:2]
    wh = pred[:, 2:4]
    boxes = jnp.concatenate([xy - wh / 2.0, xy + wh / 2.0], axis=1)
    vals = jnp.concatenate([boxes, conf, cls], axis=1)
    valsp = jnp.concatenate(
        [vals, jnp.zeros((_NP - _N, 6), jnp.float32)], axis=0)

    tb = []
    for kb in range(_KP // _C):
        ki = float(kb * _C) + lax.broadcasted_iota(jnp.int32, (_C, _NP), 0).astype(jnp.float32)
        p = (rank_row == ki).astype(jnp.float32)
        tb.append(jnp.dot(p, valsp, precision=_HI))
    topb = jnp.concatenate(tb, axis=0)
    topb_ref[0] = topb

    clsc = topb[:, 5:6]
    off = clsc * 4096.0
    x1c = topb[:, 0:1] + off
    y1c = topb[:, 1:2] + off
    x2c = topb[:, 2:3] + off
    y2c = topb[:, 3:4] + off
    areac = (x2c - x1c) * (y2c - y1c)
    x1r, y1r = _row_of(x1c), _row_of(y1c)
    x2r, y2r = _row_of(x2c), _row_of(y2c)
    arear = _row_of(areac)
    pack = _pack_matrix()

    for mb in range(_KP // _C):
        s = mb * _C
        x1b = lax.slice(x1c, (s, 0), (s + _C, 1))
        y1b = lax.slice(y1c, (s, 0), (s + _C, 1))
        x2b = lax.slice(x2c, (s, 0), (s + _C, 1))
        y2b = lax.slice(y2c, (s, 0), (s + _C, 1))
        arb = lax.slice(areac, (s, 0), (s + _C, 1))
        ltx = jnp.maximum(x1b, x1r)
        lty = jnp.maximum(y1b, y1r)
        rbx = jnp.minimum(x2b, x2r)
        rby = jnp.minimum(y2b, y2r)
        w = jnp.clip(rbx - ltx, 0.0)
        h = jnp.clip(rby - lty, 0.0)
        inter = w * h
        iou = inter / (arb + arear - inter + 1e-7)
        jl = lax.broadcasted_iota(jnp.int32, (_C, _KP), 1).astype(jnp.float32)
        ii = float(s) + lax.broadcasted_iota(jnp.int32, (_C, _KP), 0).astype(jnp.float32)
        mbv = ((iou > _IOU_T) & (jl > ii)).astype(jnp.float32)
        mp_ref[0, s:s + _C, :] = jnp.dot(mbv, pack, precision=_HI)

    topvc = topb[:, 4:5]
    topv_row = _row_of(topvc)
    klane = lax.broadcasted_iota(jnp.int32, (1, _KP), 1)
    keep0 = ((topv_row > 0.0) & (klane < _PRE_K)).astype(jnp.float32)
    ki_ref[0] = jnp.dot(keep0, pack, precision=_HI)


def _tc1(pred):
    return pl.pallas_call(
        _tc1_body,
        grid=(8,),
        in_specs=[pl.BlockSpec((1, _N, 85), lambda i: (i, 0, 0))],
        out_specs=[
            pl.BlockSpec((1, _KP, 6), lambda i: (i, 0, 0)),
            pl.BlockSpec((1, _KP, _W), lambda i: (i, 0, 0)),
            pl.BlockSpec((1, 1, _W), lambda i: (i, 0, 0)),
        ],
        out_shape=[
            jax.ShapeDtypeStruct((8, _KP, 6), jnp.float32),
            jax.ShapeDtypeStruct((8, _KP, _W), jnp.float32),
            jax.ShapeDtypeStruct((8, 1, _W), jnp.float32),
        ],
        scratch_shapes=[pltpu.VMEM((_NP, 1), jnp.float32),
                        pltpu.VMEM((_NP, 1), jnp.float32),
                        pltpu.VMEM((_NP, _C), jnp.int32)],
    )(pred)


def _sc_nms(mp, ki):
    mesh = plsc.VectorSubcoreMesh(core_axis_name="c", subcore_axis_name="s")

    @functools.partial(
        pl.kernel,
        mesh=mesh,
        out_type=jax.ShapeDtypeStruct((8, _KP), jnp.float32),
        scratch_types=[
            pltpu.VMEM((_KP * _W,), jnp.float32),
            pltpu.VMEM((_W + 16,), jnp.int32),
            pltpu.VMEM((_W,), jnp.float32),
            pltpu.VMEM((_KP,), jnp.float32),
        ],
    )
    def nms(mp_hbm, ki_hbm, keep_hbm, m_v, kw_v, kf_v, ko_v):
        cid = lax.axis_index("c")
        sid = lax.axis_index("s")
        wid = sid * 2 + cid
        lane16 = lax.broadcasted_iota(jnp.int32, (16,), 0)

        @pl.when(wid < 8)
        def _():
            pltpu.sync_copy(mp_hbm.at[wid], m_v)
            pltpu.sync_copy(ki_hbm.at[wid], kf_v)
            for w4 in range(_W // 16):
                kw_v[pl.ds(w4 * 16, 16)] = kf_v[pl.ds(w4 * 16, 16)].astype(jnp.int32)

            def step(i, carry):
                kv = kw_v[pl.ds(i >> 4, 16)]
                bit = (kv[0] >> (i & 15)) & 1

                @pl.when(bit != 0)
                def _s():
                    base = i * _W
                    for w4 in range(_W // 16):
                        mrow = m_v[pl.ds(base + w4 * 16, 16)].astype(jnp.int32)
                        kw_v[pl.ds(w4 * 16, 16)] = (
                            kw_v[pl.ds(w4 * 16, 16)] & (~mrow))

                return carry

            lax.fori_loop(0, _PRE_K, step, 0)

            for w in range(_W):
                word = kw_v[pl.ds(w, 16)][0]
                wv = jnp.full((16,), word, jnp.int32)
                ko_v[pl.ds(w * 16, 16)] = ((wv >> lane16) & 1).astype(jnp.float32)
            pltpu.sync_copy(ko_v, keep_hbm.at[wid])

    return nms(mp, ki)


def _tc2_body(topb_ref, keep_ref, out_ref, cc_ref, rk_ref, ac_ref):
    topb = topb_ref[0]
    keep = keep_ref[0]
    clsc = topb[:, 5:6]
    topvc = topb[:, 4:5]
    topv_row = _row_of(topvc)
    fsr = keep * topv_row
    fsc = _col_of(fsr)
    cc_ref[0:_KP, :] = fsc
    _ranks_looped(cc_ref, rk_ref, ac_ref, _KP)
    frank_row = _row_of(rk_ref[0:_KP, :])
    vals_k = jnp.concatenate([topb[:, 0:4], fsc, clsc], axis=1)
    outs = []
    for fb in range(3):
        ki = float(fb * _C) + lax.broadcasted_iota(jnp.int32, (_C, _KP), 0).astype(jnp.float32)
        p = (frank_row == ki).astype(jnp.float32)
        oc = jnp.dot(p, vals_k, precision=_HI)
        fvc = jnp.dot(p, fsc, precision=_HI)
        outs.append(oc * (fvc > 0.0).astype(jnp.float32))
    res = jnp.concatenate(outs, axis=0)
    out_ref[0] = lax.slice(res, (0, 0), (_MAXD, 6))


def _tc2(topb, keep):
    return pl.pallas_call(
        _tc2_body,
        grid=(8,),
        in_specs=[pl.BlockSpec((1, _KP, 6), lambda i: (i, 0, 0)),
                  pl.BlockSpec((1, 1, _KP), lambda i: (i, 0, 0))],
        out_specs=pl.BlockSpec((1, _MAXD, 6), lambda i: (i, 0, 0)),
        out_shape=jax.ShapeDtypeStruct((8, _MAXD, 6), jnp.float32),
        scratch_shapes=[pltpu.VMEM((_KP, 1), jnp.float32),
                        pltpu.VMEM((_KP, 1), jnp.float32),
                        pltpu.VMEM((_KP, _C), jnp.int32)],
    )(topb, keep)


def kernel(pred):
    topb, mp, ki = _tc1(pred)
    keep = _sc_nms(jnp.reshape(mp, (8, _KP * _W)), jnp.reshape(ki, (8, _W)))
    return _tc2(topb, jnp.reshape(keep, (8, 1, _KP)))
